# baseline (speedup 1.0000x reference)
import functools

import jax
import jax.numpy as jnp
from jax import lax
from jax.experimental import pallas as pl
from jax.experimental.pallas import tpu as pltpu
from jax.experimental.pallas import tpu_sc as plsc

E = 128
C = 5
B = 16384
NC = 2
NS = 16
NW = NC * NS
BPW = B // NW
CB = 16
NCHUNK = BPW // CB
GRP = 2
NGROUP = NCHUNK // GRP
LANES = 8


def _w2v_body(tgt_hbm, ctx_hbm, ttab_hbm, ctab_hbm, out_hbm,
              tidx_v, cidx_v, wbuf, cbuf, out_v,
              sem_w0, sem_c0, sem_w1, sem_c1):
    wid = lax.axis_index("s") * NC + lax.axis_index("c")
    base = wid * BPW

    pltpu.sync_copy(tgt_hbm.at[pl.ds(base, BPW)], tidx_v)
    pltpu.sync_copy(ctx_hbm.at[pl.ds(base * C, BPW * C)], cidx_v)

    sems = ((sem_w0, sem_c0), (sem_w1, sem_c1))
    lane = lax.iota(jnp.int32, 16)

    def start_group(g, parity):
        sw, sc = sems[parity]
        for b in range(GRP):
            koff = pl.multiple_of((g * GRP + b) * CB, 8)
            slot = parity * GRP + b
            pltpu.make_async_copy(
                ttab_hbm.at[tidx_v.at[pl.ds(koff, CB)]], wbuf.at[slot], sw
            ).start()
            pltpu.make_async_copy(
                ctab_hbm.at[cidx_v.at[pl.ds(koff * C, CB * C)]], cbuf.at[slot], sc
            ).start()

    def wait_group(parity):
        sw, sc = sems[parity]
        for b in range(GRP):
            slot = parity * GRP + b
            pltpu.make_async_copy(ttab_hbm.at[pl.ds(0, CB)], wbuf.at[slot], sw).wait()
            pltpu.make_async_copy(ctab_hbm.at[pl.ds(0, CB * C)], cbuf.at[slot], sc).wait()

    def compute_group(g, parity):
        def slot_body(s4, _):
            k = g * GRP + s4
            slot = parity * GRP + s4
            res = [jnp.zeros((16,), jnp.float32) for _ in range(C)]
            for j in range(CB):
                w = [wbuf[slot, j, pl.ds(16 * t, 16)] for t in range(LANES)]
                for c in range(C):
                    r = j * C + c
                    acc = w[0] * cbuf[slot, r, pl.ds(0, 16)]
                    for t in range(1, LANES):
                        acc = acc + w[t] * cbuf[slot, r, pl.ds(16 * t, 16)]
                    res[r // 16] = jnp.where(lane == (r % 16), jnp.sum(acc),
                                             res[r // 16])
            off = pl.multiple_of(k * CB * C, 16)
            for i in range(C):
                out_v[pl.ds(off + 16 * i, 16)] = res[i]
            return _

        lax.fori_loop(0, GRP, slot_body, None)

    start_group(0, 0)

    def group_pair(i, _):
        g = 2 * i

        @pl.when(g + 1 < NGROUP)
        def _pf1():
            start_group(g + 1, 1)

        wait_group(0)
        compute_group(g, 0)

        @pl.when(g + 2 < NGROUP)
        def _pf2():
            start_group(g + 2, 0)

        wait_group(1)
        compute_group(g + 1, 1)
        return _

    lax.fori_loop(0, NGROUP // 2, group_pair, None)
    pltpu.sync_copy(out_v, out_hbm.at[pl.ds(base * C, BPW * C)])


@jax.jit
def _w2v(tgt, ctx, ttab, ctab):
    mesh = plsc.VectorSubcoreMesh(core_axis_name="c", subcore_axis_name="s")
    f = functools.partial(
        pl.kernel,
        out_type=jax.ShapeDtypeStruct((B * C,), jnp.float32),
        mesh=mesh,
        compiler_params=pltpu.CompilerParams(needs_layout_passes=False),
        scratch_types=[
            pltpu.VMEM((BPW,), jnp.int32),
            pltpu.VMEM((BPW * C,), jnp.int32),
            pltpu.VMEM((2 * GRP, CB, E), jnp.float32),
            pltpu.VMEM((2 * GRP, CB * C, E), jnp.float32),
            pltpu.VMEM((BPW * C,), jnp.float32),
            pltpu.SemaphoreType.DMA,
            pltpu.SemaphoreType.DMA,
            pltpu.SemaphoreType.DMA,
            pltpu.SemaphoreType.DMA,
        ],
    )(_w2v_body)
    return f(tgt, ctx, ttab, ctab)


def kernel(target, context, target_table, context_table):
    if target.ndim == 2:
        target = jnp.squeeze(target, axis=1)
    ctx = context.reshape(B * C)
    out = _w2v(target, ctx, target_table, context_table)
    return out.reshape(B, C)


# speedup vs baseline: 1.4755x; 1.4755x over previous
import functools

import jax
import jax.numpy as jnp
from jax import lax
from jax.experimental import pallas as pl
from jax.experimental.pallas import tpu as pltpu
from jax.experimental.pallas import tpu_sc as plsc

E = 128
C = 5
B = 16384
NC = 2
NS = 16
NW = NC * NS
BPW = B // NW
CB = 16
NCHUNK = BPW // CB
NBUF = 4
LANES = 8


def _w2v_body(tgt_hbm, ctx_hbm, ttab_hbm, ctab_hbm, out_hbm,
              tidx_v, cidx_v, wbuf, cbuf, out_v, *sems):
    wid = lax.axis_index("s") * NC + lax.axis_index("c")
    base = wid * BPW

    pltpu.sync_copy(tgt_hbm.at[pl.ds(base, BPW)], tidx_v)
    pltpu.sync_copy(ctx_hbm.at[pl.ds(base * C, BPW * C)], cidx_v)

    def start(k, slot):
        sw, sc = sems[2 * slot], sems[2 * slot + 1]
        koff = pl.multiple_of(k * CB, 8)
        pltpu.make_async_copy(
            ttab_hbm.at[tidx_v.at[pl.ds(koff, CB)]], wbuf.at[slot], sw
        ).start()
        pltpu.make_async_copy(
            ctab_hbm.at[cidx_v.at[pl.ds(koff * C, CB * C)]], cbuf.at[slot], sc
        ).start()

    def wait(slot):
        sw, sc = sems[2 * slot], sems[2 * slot + 1]
        pltpu.make_async_copy(ttab_hbm.at[pl.ds(0, CB)], wbuf.at[slot], sw).wait()
        pltpu.make_async_copy(ctab_hbm.at[pl.ds(0, CB * C)], cbuf.at[slot], sc).wait()

    lane = lax.iota(jnp.int32, 16)

    def compute(k, slot):
        def row_body(j, _):
            w = [wbuf[slot, j, pl.ds(16 * t, 16)] for t in range(LANES)]
            vec = jnp.zeros((16,), jnp.float32)
            for c in range(C):
                r = j * C + c
                acc = w[0] * cbuf[slot, r, pl.ds(0, 16)]
                for t in range(1, LANES):
                    acc = acc + w[t] * cbuf[slot, r, pl.ds(16 * t, 16)]
                vec = jnp.where(lane == c, jnp.sum(acc), vec)
            out_v[k * CB + j, pl.ds(0, 16)] = vec
            return _

        lax.fori_loop(0, CB, row_body, None)

    for s in range(NBUF - 1):
        start(s, s)

    def group_body(g, _):
        for b in range(NBUF):
            k = g * NBUF + b

            @pl.when(k + NBUF - 1 < NCHUNK)
            def _prefetch():
                start(k + NBUF - 1, (b + NBUF - 1) % NBUF)

            wait(b)
            compute(k, b)
        return _

    lax.fori_loop(0, NCHUNK // NBUF, group_body, None)
    pltpu.sync_copy(out_v, out_hbm.at[wid])


@jax.jit
def _w2v(tgt, ctx, ttab, ctab):
    mesh = plsc.VectorSubcoreMesh(core_axis_name="c", subcore_axis_name="s")
    f = functools.partial(
        pl.kernel,
        out_type=jax.ShapeDtypeStruct((NW, BPW, 16), jnp.float32),
        mesh=mesh,
        compiler_params=pltpu.CompilerParams(needs_layout_passes=False),
        scratch_types=[
            pltpu.VMEM((BPW,), jnp.int32),
            pltpu.VMEM((BPW * C,), jnp.int32),
            pltpu.VMEM((NBUF, CB, E), jnp.float32),
            pltpu.VMEM((NBUF, CB * C, E), jnp.float32),
            pltpu.VMEM((BPW, 16), jnp.float32),
        ] + [pltpu.SemaphoreType.DMA] * (2 * NBUF),
    )(_w2v_body)
    return f(tgt, ctx, ttab, ctab)


def kernel(target, context, target_table, context_table):
    if target.ndim == 2:
        target = jnp.squeeze(target, axis=1)
    ctx = context.reshape(B * C)
    out = _w2v(target, ctx, target_table, context_table)
    return out.reshape(B, 16)[:, :C]


# speedup vs baseline: 2.3170x; 1.5703x over previous
import functools

import jax
import jax.numpy as jnp
from jax import lax
from jax.experimental import pallas as pl
from jax.experimental.pallas import tpu as pltpu
from jax.experimental.pallas import tpu_sc as plsc

E = 128
C = 5
B = 16384
NC = 2
NS = 16
NW = NC * NS
BPW = B // NW
CB = 16
NCHUNK = BPW // CB
NBUF = 4
LANES = 8


def _w2v_body(tgt_hbm, ctx_hbm, ttab_hbm, ctab_hbm, out_hbm,
              tidx_v, cidx_v, wbuf, cbuf, out_v, *sems):
    wid = lax.axis_index("s") * NC + lax.axis_index("c")
    base = wid * BPW

    pltpu.sync_copy(tgt_hbm.at[pl.ds(base, BPW)], tidx_v)
    pltpu.sync_copy(ctx_hbm.at[:, pl.ds(base, BPW)], cidx_v)

    def start(k, slot):
        sw, sc = sems[2 * slot], sems[2 * slot + 1]
        koff = pl.multiple_of(k * CB, 8)
        pltpu.make_async_copy(
            ttab_hbm.at[tidx_v.at[pl.ds(koff, CB)]], wbuf.at[slot], sw
        ).start()
        for c in range(C):
            pltpu.make_async_copy(
                ctab_hbm.at[cidx_v.at[c, pl.ds(koff, CB)]],
                cbuf.at[slot, pl.ds(c * CB, CB)], sc
            ).start()

    def wait(slot):
        sw, sc = sems[2 * slot], sems[2 * slot + 1]
        pltpu.make_async_copy(ttab_hbm.at[pl.ds(0, CB)], wbuf.at[slot], sw).wait()
        for c in range(C):
            pltpu.make_async_copy(
                ttab_hbm.at[pl.ds(0, CB)], cbuf.at[slot, pl.ds(c * CB, CB)], sc
            ).wait()

    lane = lax.iota(jnp.int32, 16)

    def compute(k, slot):
        def row_body(j, vecs):
            w = [wbuf[slot, j, pl.ds(16 * t, 16)] for t in range(LANES)]
            out = []
            for c in range(C):
                r = c * CB + j
                acc = w[0] * cbuf[slot, r, pl.ds(0, 16)]
                for t in range(1, LANES):
                    acc = acc + w[t] * cbuf[slot, r, pl.ds(16 * t, 16)]
                out.append(jnp.where(lane == j, jnp.sum(acc), vecs[c]))
            return tuple(out)

        init = tuple(jnp.zeros((16,), jnp.float32) for _ in range(C))
        vecs = lax.fori_loop(0, CB, row_body, init)
        koff = pl.multiple_of(k * CB, 16)
        for c in range(C):
            out_v[c, pl.ds(koff, 16)] = vecs[c]

    for s in range(NBUF - 1):
        start(s, s)

    def group_body(g, _):
        for b in range(NBUF):
            k = g * NBUF + b

            @pl.when(k + NBUF - 1 < NCHUNK)
            def _prefetch():
                start(k + NBUF - 1, (b + NBUF - 1) % NBUF)

            wait(b)
            compute(k, b)
        return _

    lax.fori_loop(0, NCHUNK // NBUF, group_body, None)
    pltpu.sync_copy(out_v, out_hbm.at[:, pl.ds(base, BPW)])


@jax.jit
def _w2v(tgt, ctx_t, ttab, ctab):
    mesh = plsc.VectorSubcoreMesh(core_axis_name="c", subcore_axis_name="s")
    f = functools.partial(
        pl.kernel,
        out_type=jax.ShapeDtypeStruct((C, B), jnp.float32),
        mesh=mesh,
        compiler_params=pltpu.CompilerParams(needs_layout_passes=False),
        scratch_types=[
            pltpu.VMEM((BPW,), jnp.int32),
            pltpu.VMEM((C, BPW), jnp.int32),
            pltpu.VMEM((NBUF, CB, E), jnp.float32),
            pltpu.VMEM((NBUF, CB * C, E), jnp.float32),
            pltpu.VMEM((C, BPW), jnp.float32),
        ] + [pltpu.SemaphoreType.DMA] * (2 * NBUF),
    )(_w2v_body)
    return f(tgt, ctx_t, ttab, ctab)


def kernel(target, context, target_table, context_table):
    if target.ndim == 2:
        target = jnp.squeeze(target, axis=1)
    out_t = _w2v(target, context.T, target_table, context_table)
    return out_t.T
